# baseline (speedup 1.0000x reference)
import functools

import jax
import jax.numpy as jnp
from jax import lax
from jax.experimental import pallas as pl
from jax.experimental.pallas import tpu as pltpu
from jax.experimental.pallas import tpu_sc as plsc


@functools.cache
def _build(B, V, D):
    info = plsc.get_sparse_core_info()
    NC, NS, L = info.num_cores, info.num_subcores, info.num_lanes
    NW = NC * NS
    assert D % NW == 0
    DPW = D // NW
    CH = 4096
    NCHUNK = B // CH
    assert B % CH == 0
    UNROLL = 8
    assert CH % (L * UNROLL) == 0

    @functools.partial(
        pl.kernel,
        mesh=plsc.VectorSubcoreMesh(core_axis_name="c", subcore_axis_name="s"),
        compiler_params=pltpu.CompilerParams(needs_layout_passes=False),
        out_type=jax.ShapeDtypeStruct((D, B), jnp.float32),
        scratch_types=[
            pltpu.VMEM((V,), jnp.float32),
            pltpu.VMEM((2, CH), jnp.int32),
            pltpu.VMEM((2, CH), jnp.float32),
            pltpu.SemaphoreType.DMA,
            pltpu.SemaphoreType.DMA,
            pltpu.SemaphoreType.DMA,
            pltpu.SemaphoreType.DMA,
            pltpu.SemaphoreType.DMA,
        ],
    )
    def gather_kernel(
        labels_hbm, tableT_hbm, outT_hbm,
        row_v, lab_v, out_v,
        sem_row, sem_lab0, sem_lab1, sem_out0, sem_out1,
    ):
        wid = lax.axis_index("s") * NC + lax.axis_index("c")
        sem_lab = (sem_lab0, sem_lab1)
        sem_out = (sem_out0, sem_out1)

        lab_pending = {}
        out_pending = {0: None, 1: None}

        def fetch_labels(chunk, buf):
            lab_pending[buf] = pltpu.async_copy(
                labels_hbm.at[pl.ds(chunk * CH, CH)], lab_v.at[buf], sem_lab[buf]
            )

        fetch_labels(0, 0)
        row_cp = pltpu.async_copy(tableT_hbm.at[wid * DPW], row_v, sem_row)

        for t in range(DPW):
            j = wid * DPW + t
            row_cp.wait()
            for c in range(NCHUNK):
                cb = (t * NCHUNK + c) % 2
                lab_pending.pop(cb).wait()
                if c + 1 < NCHUNK:
                    fetch_labels(c + 1, 1 - cb)
                elif t + 1 < DPW:
                    fetch_labels(0, 1 - cb)
                if out_pending[cb] is not None:
                    out_pending[cb].wait()

                def body(g, _, cb=cb):
                    local = g * (L * UNROLL)
                    for u in range(UNROLL):
                        off = local + u * L
                        idx = lab_v[cb, pl.ds(off, L)]
                        vals = plsc.load_gather(row_v, [idx])
                        out_v[cb, pl.ds(off, L)] = vals
                    return 0

                lax.fori_loop(0, CH // (L * UNROLL), body, 0)
                if c == NCHUNK - 1 and t + 1 < DPW:
                    row_cp = pltpu.async_copy(
                        tableT_hbm.at[j + 1], row_v, sem_row
                    )
                out_pending[cb] = pltpu.async_copy(
                    out_v.at[cb], outT_hbm.at[j, pl.ds(c * CH, CH)], sem_out[cb]
                )

        for buf in (0, 1):
            if out_pending[buf] is not None:
                out_pending[buf].wait()

    return gather_kernel


def kernel(labels, train, table):
    (B,) = labels.shape
    V, D = table.shape
    gather_kernel = _build(B, V, D)
    outT = gather_kernel(labels.astype(jnp.int32), table.T)
    return outT.T


# speedup vs baseline: 1.2091x; 1.2091x over previous
import functools

import jax
import jax.numpy as jnp
from jax import lax
from jax.experimental import pallas as pl
from jax.experimental.pallas import tpu as pltpu
from jax.experimental.pallas import tpu_sc as plsc


@functools.cache
def _build(B, V, D):
    info = plsc.get_sparse_core_info()
    NC, NS, L = info.num_cores, info.num_subcores, info.num_lanes
    NW = NC * NS
    assert D % NW == 0
    DPW = D // NW
    HB = B // 2
    UNROLL = 8
    assert HB % (L * UNROLL) == 0

    @functools.partial(
        pl.kernel,
        mesh=plsc.VectorSubcoreMesh(core_axis_name="c", subcore_axis_name="s"),
        compiler_params=pltpu.CompilerParams(needs_layout_passes=False),
        out_type=jax.ShapeDtypeStruct((D, B), jnp.float32),
        scratch_types=[
            pltpu.VMEM((B,), jnp.int32),
            pltpu.VMEM((V,), jnp.float32),
            pltpu.VMEM((HB,), jnp.float32),
        ],
    )
    def gather_kernel(labels_hbm, tableT_hbm, outT_hbm, lab_v, row_v, out_v):
        wid = lax.axis_index("s") * NC + lax.axis_index("c")
        pltpu.sync_copy(labels_hbm, lab_v)
        for t in range(DPW):
            j = wid * DPW + t
            pltpu.sync_copy(tableT_hbm.at[j], row_v)
            for h in range(B // HB):

                @plsc.parallel_loop(0, HB, step=L, unroll=UNROLL)
                def body(off, h=h):
                    idx = lab_v[pl.ds(h * HB + off, L)]
                    vals = plsc.load_gather(row_v, [idx])
                    out_v[pl.ds(off, L)] = vals

                pltpu.sync_copy(out_v, outT_hbm.at[j, pl.ds(h * HB, HB)])

    return gather_kernel


def kernel(labels, train, table):
    (B,) = labels.shape
    V, D = table.shape
    gather_kernel = _build(B, V, D)
    outT = gather_kernel(labels.astype(jnp.int32), table.T)
    return outT.T


# speedup vs baseline: 1.2261x; 1.0140x over previous
import functools

import jax
import jax.numpy as jnp
from jax import lax
from jax.experimental import pallas as pl
from jax.experimental.pallas import tpu as pltpu
from jax.experimental.pallas import tpu_sc as plsc


@functools.cache
def _build(B, V, D):
    info = plsc.get_sparse_core_info()
    NC, NS, L = info.num_cores, info.num_subcores, info.num_lanes
    NW = NC * NS
    assert D % NW == 0
    DPW = D // NW
    CH = 4096
    NCHUNK = B // CH
    assert B % CH == 0
    UNROLL = 8
    assert CH % (L * UNROLL) == 0

    @functools.partial(
        pl.kernel,
        mesh=plsc.VectorSubcoreMesh(core_axis_name="c", subcore_axis_name="s"),
        compiler_params=pltpu.CompilerParams(needs_layout_passes=False),
        out_type=jax.ShapeDtypeStruct((D, B), jnp.float32),
        scratch_types=[
            pltpu.VMEM((B,), jnp.int32),
            pltpu.VMEM((V,), jnp.float32),
            pltpu.VMEM((2, CH), jnp.float32),
            pltpu.SemaphoreType.DMA,
            pltpu.SemaphoreType.DMA,
            pltpu.SemaphoreType.DMA,
            pltpu.SemaphoreType.DMA,
        ],
    )
    def gather_kernel(
        labels_hbm, tableT_hbm, outT_hbm,
        lab_v, row_v, out_v,
        sem_lab, sem_row, sem_out0, sem_out1,
    ):
        wid = lax.axis_index("s") * NC + lax.axis_index("c")
        sem_out = (sem_out0, sem_out1)

        lab_cp = pltpu.async_copy(labels_hbm, lab_v, sem_lab)
        row_cp = pltpu.async_copy(tableT_hbm.at[wid * DPW], row_v, sem_row)
        lab_cp.wait()

        out_pending = {0: None, 1: None}
        for t in range(DPW):
            j = wid * DPW + t
            row_cp.wait()
            for c in range(NCHUNK):
                cb = (t * NCHUNK + c) % 2
                if out_pending[cb] is not None:
                    out_pending[cb].wait()

                @plsc.parallel_loop(0, CH, step=L, unroll=UNROLL)
                def body(off, c=c, cb=cb):
                    idx = lab_v[pl.ds(c * CH + off, L)]
                    vals = plsc.load_gather(row_v, [idx])
                    out_v[cb, pl.ds(off, L)] = vals

                if c == NCHUNK - 1 and t + 1 < DPW:
                    row_cp = pltpu.async_copy(
                        tableT_hbm.at[j + 1], row_v, sem_row
                    )
                out_pending[cb] = pltpu.async_copy(
                    out_v.at[cb], outT_hbm.at[j, pl.ds(c * CH, CH)], sem_out[cb]
                )

        for buf in (0, 1):
            if out_pending[buf] is not None:
                out_pending[buf].wait()

    return gather_kernel


def kernel(labels, train, table):
    (B,) = labels.shape
    V, D = table.shape
    gather_kernel = _build(B, V, D)
    outT = gather_kernel(labels.astype(jnp.int32), table.T)
    return outT.T
